# Initial kernel scaffold; baseline (speedup 1.0000x reference)
import jax
import jax.numpy as jnp
from jax.experimental import pallas as pl


def kernel(X, mask, W, b, gamma, beta):
    raise NotImplementedError("write your pallas kernel here")


# speedup vs baseline: 1.2983x; 1.2983x over previous
import functools

import numpy as np
import jax
import jax.numpy as jnp
from jax import lax
from jax.experimental import pallas as pl
from jax.experimental.pallas import tpu as pltpu

K_NEIGH = 30
NUM_PE = 16
NUM_RBF = 16


def _normalize(x, axis=-1, eps=1e-12):
    n = jnp.linalg.norm(x, axis=axis, keepdims=True)
    return x / jnp.maximum(n, eps)


def _build_o9(X):
    B, N, _ = X.shape
    dX = X[:, 1:, :] - X[:, -1:, :]
    U = _normalize(dX, -1)
    u_2 = U[:, :-2, :]
    u_1 = U[:, 1:-1, :]
    n_2 = _normalize(jnp.cross(u_2, u_1), -1)
    o_1 = _normalize(u_2 - u_1, -1)
    O = jnp.stack([o_1, n_2, jnp.cross(o_1, n_2)], -1)
    O = O.reshape(B, N - 3, 9)
    return jnp.pad(O, ((0, 0), (1, 2), (0, 0)))


def _knn_body(x_ref, xt_ref, idx_ref, dn_ref, ds, *, R, N):
    xq = x_ref[0]
    xt = xt_ref[0]
    dx = xq[:, 0:1] - xt[0:1, :]
    dy = xq[:, 1:2] - xt[1:2, :]
    dz = xq[:, 2:3] - xt[2:3, :]
    s = dx * dx + dy * dy + dz * dz
    ds[...] = jnp.sqrt(s + 1e-6)
    iota = lax.broadcasted_iota(jnp.int32, (R, N), 1)
    for j in range(K_NEIGH):
        d = ds[...]
        m = jnp.min(d, axis=1, keepdims=True)
        sel = d == m
        iv = jnp.min(jnp.where(sel, iota, N), axis=1, keepdims=True)
        idx_ref[0, :, j:j + 1] = iv
        dn_ref[0, :, j:j + 1] = m
        ds[...] = jnp.where(iota == iv, jnp.float32(jnp.inf), d)


def _knn(X):
    B, N, _ = X.shape
    R = 32 if N % 32 == 0 else 8
    XT = jnp.swapaxes(X, 1, 2)
    body = functools.partial(_knn_body, R=R, N=N)
    return pl.pallas_call(
        body,
        grid=(B, N // R),
        in_specs=[
            pl.BlockSpec((1, R, 3), lambda b, n: (b, n, 0)),
            pl.BlockSpec((1, 3, N), lambda b, n: (b, 0, 0)),
        ],
        out_specs=[
            pl.BlockSpec((1, R, K_NEIGH), lambda b, n: (b, n, 0)),
            pl.BlockSpec((1, R, K_NEIGH), lambda b, n: (b, n, 0)),
        ],
        out_shape=[
            jax.ShapeDtypeStruct((B, N, K_NEIGH), jnp.int32),
            jax.ShapeDtypeStruct((B, N, K_NEIGH), jnp.float32),
        ],
        scratch_shapes=[pltpu.VMEM((R, N), jnp.float32)],
    )(X, XT)


def _np_g(rows, cols, pairs):
    g = np.zeros((rows, cols), np.float32)
    for r, c in pairs:
        g[r, c] = 1.0
    return g

_G9 = _np_g(9, 3, [(j, j // 3) for j in range(9)])
_G27 = _np_g(27, 9, [(9 * c + 3 * a + b, 3 * a + b)
                     for c in range(3) for a in range(3) for b in range(3)])
_C9 = np.zeros((9, 7), np.float32)
_C9[0, 0], _C9[4, 0], _C9[8, 0] = 1, -1, -1
_C9[0, 1], _C9[4, 1], _C9[8, 1] = -1, 1, -1
_C9[0, 2], _C9[4, 2], _C9[8, 2] = -1, -1, 1
_C9[7, 3], _C9[5, 3] = 1, -1
_C9[2, 4], _C9[6, 4] = 1, -1
_C9[3, 5], _C9[1, 5] = 1, -1
_C9[0, 6] = _C9[4, 6] = _C9[8, 6] = 1

_FREQ = np.exp(np.arange(0, NUM_PE, 2, dtype=np.float32)
               * (-(np.log(10000.0) / NUM_PE))).reshape(1, NUM_PE // 2)
_RBF_MU = np.linspace(0.0, 20.0, NUM_RBF, dtype=np.float32).reshape(1, NUM_RBF)
_RBF_SIG = (20.0 - 0.0) / NUM_RBF


def _edge_body(nf_ref, qf_ref, dpe_ref, dn_ref, wt_ref, b_ref, g_ref, bt_ref,
               g9_ref, g27_ref, c9_ref, freq_ref, rbfmu_ref, out_ref, *, E):
    nf = nf_ref[...]
    qf = qf_ref[...]
    xi = qf[:, 0:3]
    oi = qf[:, 3:12]
    xn = nf[:, 0:3]
    on = nf[:, 3:12]
    dxn = xn - xi
    p9 = oi * jnp.concatenate([dxn, dxn, dxn], axis=1)
    du_raw = jnp.dot(p9, g9_ref[...], preferred_element_type=jnp.float32)
    dun = jnp.sqrt(jnp.sum(du_raw * du_raw, axis=1, keepdims=True))
    du = du_raw / jnp.maximum(dun, 1e-12)
    a27 = jnp.concatenate(
        [jnp.broadcast_to(oi[:, j:j + 1], (E, 3)) for j in range(9)], axis=1)
    b27 = jnp.concatenate(
        [on[:, 3 * c:3 * c + 3] for c in range(3) for _ in range(3)], axis=1)
    r9 = jnp.dot(a27 * b27, g27_ref[...], preferred_element_type=jnp.float32)
    t = jnp.dot(r9, c9_ref[...], preferred_element_type=jnp.float32)
    mag = 0.5 * jnp.sqrt(jnp.abs(1.0 + t[:, 0:3]))
    xyz = jnp.sign(t[:, 3:6]) * mag
    w = jnp.sqrt(jnp.maximum(1.0 + t[:, 6:7], 0.0)) / 2.0
    q = jnp.concatenate([xyz, w], axis=1)
    qn = jnp.sqrt(jnp.sum(q * q, axis=1, keepdims=True))
    q = q / jnp.maximum(qn, 1e-12)
    ang = dpe_ref[...] * freq_ref[...]
    rb = (dn_ref[...] - rbfmu_ref[...]) / _RBF_SIG
    rbf = jnp.exp(-(rb * rb))
    f = jnp.concatenate(
        [jnp.cos(ang), jnp.sin(ang), rbf, du, q, jnp.zeros((E, 1), jnp.float32)],
        axis=1)
    e = jnp.dot(f, wt_ref[...], preferred_element_type=jnp.float32) + b_ref[...]
    mu = jnp.mean(e, axis=1, keepdims=True)
    var = jnp.mean((e - mu) * (e - mu), axis=1, keepdims=True)
    out_ref[...] = g_ref[...] * (e - mu) / jnp.sqrt(var + 1e-6) + bt_ref[...]


def _edges(nf, qf, dpe, dn, Wt, b, gamma, beta):
    BNK = nf.shape[0]
    E = 512 if BNK % 512 == 0 else 480
    D = Wt.shape[1]
    body = functools.partial(_edge_body, E=E)
    return pl.pallas_call(
        body,
        grid=(BNK // E,),
        in_specs=[
            pl.BlockSpec((E, 16), lambda i: (i, 0)),
            pl.BlockSpec((E, 16), lambda i: (i, 0)),
            pl.BlockSpec((E, 1), lambda i: (i, 0)),
            pl.BlockSpec((E, 1), lambda i: (i, 0)),
            pl.BlockSpec((40, D), lambda i: (0, 0)),
            pl.BlockSpec((1, D), lambda i: (0, 0)),
            pl.BlockSpec((1, D), lambda i: (0, 0)),
            pl.BlockSpec((1, D), lambda i: (0, 0)),
            pl.BlockSpec((9, 3), lambda i: (0, 0)),
            pl.BlockSpec((27, 9), lambda i: (0, 0)),
            pl.BlockSpec((9, 7), lambda i: (0, 0)),
            pl.BlockSpec((1, NUM_PE // 2), lambda i: (0, 0)),
            pl.BlockSpec((1, NUM_RBF), lambda i: (0, 0)),
        ],
        out_specs=pl.BlockSpec((E, D), lambda i: (i, 0)),
        out_shape=jax.ShapeDtypeStruct((BNK, D), jnp.float32),
    )(nf, qf, dpe, dn, Wt, b, gamma, beta,
      jnp.asarray(_G9), jnp.asarray(_G27), jnp.asarray(_C9),
      jnp.asarray(_FREQ), jnp.asarray(_RBF_MU))


def kernel(X, mask, W, b, gamma, beta):
    B, N, _ = X.shape
    K = K_NEIGH
    D = W.shape[0]
    E_idx, Dn = _knn(X)
    O9 = _build_o9(X)
    tbl = jnp.concatenate(
        [X, O9, jnp.zeros((B, N, 4), jnp.float32)], axis=-1)
    flat_idx = E_idx.reshape(B, N * K)
    nf = jnp.take_along_axis(tbl, flat_idx[:, :, None], axis=1)
    nf = nf.reshape(B * N * K, 16)
    qf = jnp.broadcast_to(tbl[:, :, None, :], (B, N, K, 16)).reshape(-1, 16)
    ii = jnp.arange(N, dtype=jnp.float32).reshape(1, N, 1)
    dpe = (E_idx.astype(jnp.float32) - ii).reshape(-1, 1)
    dnf = Dn.reshape(-1, 1)
    Wt = jnp.zeros((40, D), jnp.float32).at[:W.shape[1]].set(W.T)
    E_flat = _edges(nf, qf, dpe, dnf, Wt,
                    b.reshape(1, D), gamma.reshape(1, D), beta.reshape(1, D))
    return E_flat.reshape(B, N, K, D), E_idx


# speedup vs baseline: 2.1946x; 1.6904x over previous
import functools

import numpy as np
import jax
import jax.numpy as jnp
from jax import lax
from jax.experimental import pallas as pl
from jax.experimental.pallas import tpu as pltpu
from jax.experimental.pallas import tpu_sc as plsc

K_NEIGH = 30
NUM_PE = 16
NUM_RBF = 16


def _normalize(x, axis=-1, eps=1e-12):
    n = jnp.linalg.norm(x, axis=axis, keepdims=True)
    return x / jnp.maximum(n, eps)


def _build_o9(X):
    B, N, _ = X.shape
    dX = X[:, 1:, :] - X[:, -1:, :]
    U = _normalize(dX, -1)
    u_2 = U[:, :-2, :]
    u_1 = U[:, 1:-1, :]
    n_2 = _normalize(jnp.cross(u_2, u_1), -1)
    o_1 = _normalize(u_2 - u_1, -1)
    O = jnp.stack([o_1, n_2, jnp.cross(o_1, n_2)], -1)
    O = O.reshape(B, N - 3, 9)
    return jnp.pad(O, ((0, 0), (1, 2), (0, 0)))


def _knn_body(x_ref, xt_ref, idx_ref, dn_ref, *, R, N):
    xq = x_ref[0]
    xt = xt_ref[0]
    dx = xq[:, 0:1] - xt[0:1, :]
    dy = xq[:, 1:2] - xt[1:2, :]
    dz = xq[:, 2:3] - xt[2:3, :]
    s = dx * dx + dy * dy + dz * dz
    d = jnp.sqrt(s + 1e-6)
    iota = lax.broadcasted_iota(jnp.int32, (R, N), 1)
    for j in range(K_NEIGH):
        m = jnp.min(d, axis=1, keepdims=True)
        sel = d == m
        iv = jnp.min(jnp.where(sel, iota, N), axis=1, keepdims=True)
        idx_ref[0, :, j:j + 1] = iv
        dn_ref[0, :, j:j + 1] = m
        d = jnp.where(iota == iv, jnp.float32(jnp.inf), d)


def _knn(X):
    B, N, _ = X.shape
    R = 32 if N % 32 == 0 else 8
    XT = jnp.swapaxes(X, 1, 2)
    body = functools.partial(_knn_body, R=R, N=N)
    return pl.pallas_call(
        body,
        grid=(B, N // R),
        in_specs=[
            pl.BlockSpec((1, R, 3), lambda b, n: (b, n, 0)),
            pl.BlockSpec((1, 3, N), lambda b, n: (b, 0, 0)),
        ],
        out_specs=[
            pl.BlockSpec((1, R, K_NEIGH), lambda b, n: (b, n, 0)),
            pl.BlockSpec((1, R, K_NEIGH), lambda b, n: (b, n, 0)),
        ],
        out_shape=[
            jax.ShapeDtypeStruct((B, N, K_NEIGH), jnp.int32),
            jax.ShapeDtypeStruct((B, N, K_NEIGH), jnp.float32),
        ],
    )(X, XT)


_FC = 12


def _sc_gather(tbl2, idxg):
    BNK = idxg.shape[0]
    V = tbl2.shape[0] // _FC
    NC, NS = 2, 16
    NW = NC * NS
    bpw = BNK // NW
    CH = 1920 if bpw % 1920 == 0 else bpw
    NCH = bpw // CH
    NG = CH // 16
    mesh = plsc.VectorSubcoreMesh(core_axis_name="c", subcore_axis_name="s",
                                  num_cores=NC, num_subcores=NS)

    @functools.partial(
        pl.kernel, mesh=mesh,
        out_type=jax.ShapeDtypeStruct((BNK * _FC,), jnp.float32),
        compiler_params=pltpu.CompilerParams(needs_layout_passes=False),
        scratch_types=[
            pltpu.VMEM((V * _FC,), jnp.float32),
            pltpu.VMEM((CH,), jnp.int32),
            pltpu.VMEM((CH * _FC,), jnp.float32),
        ],
    )
    def k(tbl_hbm, idx_hbm, out_hbm, tbl_v, idx_v, rows_v):
        wid = lax.axis_index("s") * NC + lax.axis_index("c")
        base = wid * bpw
        pltpu.sync_copy(tbl_hbm, tbl_v)
        iota16 = lax.iota(jnp.int32, 16)

        def chunk(c, carry):
            off = base + c * CH
            pltpu.sync_copy(idx_hbm.at[pl.ds(off, CH)], idx_v)

            def group(g, carry2):
                idx16 = idx_v[pl.ds(g * 16, 16)]
                src = idx16 * _FC
                dst = (iota16 + g * 16) * _FC
                for col in range(_FC):
                    v = plsc.load_gather(tbl_v, [src + col])
                    plsc.store_scatter(rows_v, [dst + col], v)
                return carry2

            lax.fori_loop(0, NG, group, 0)
            pltpu.sync_copy(rows_v, out_hbm.at[pl.ds(off * _FC, CH * _FC)])
            return carry

        lax.fori_loop(0, NCH, chunk, 0)

    return k(tbl2, idxg)


def _np_g(rows, cols, pairs):
    g = np.zeros((rows, cols), np.float32)
    for r, c in pairs:
        g[r, c] = 1.0
    return g

_G9 = _np_g(9, 3, [(j, j // 3) for j in range(9)])
_G27 = _np_g(27, 9, [(9 * c + 3 * a + b, 3 * a + b)
                     for c in range(3) for a in range(3) for b in range(3)])
_C9 = np.zeros((9, 7), np.float32)
_C9[0, 0], _C9[4, 0], _C9[8, 0] = 1, -1, -1
_C9[0, 1], _C9[4, 1], _C9[8, 1] = -1, 1, -1
_C9[0, 2], _C9[4, 2], _C9[8, 2] = -1, -1, 1
_C9[7, 3], _C9[5, 3] = 1, -1
_C9[2, 4], _C9[6, 4] = 1, -1
_C9[3, 5], _C9[1, 5] = 1, -1
_C9[0, 6] = _C9[4, 6] = _C9[8, 6] = 1

_FREQ = np.exp(np.arange(0, NUM_PE, 2, dtype=np.float32)
               * (-(np.log(10000.0) / NUM_PE))).reshape(1, NUM_PE // 2)
_RBF_MU = np.linspace(0.0, 20.0, NUM_RBF, dtype=np.float32).reshape(1, NUM_RBF)
_RBF_SIG = (20.0 - 0.0) / NUM_RBF


def _edge_body(nf_ref, qf_ref, dpe_ref, dn_ref, wt_ref, b_ref, g_ref, bt_ref,
               g9_ref, g27_ref, c9_ref, freq_ref, rbfmu_ref, out_ref, *, E):
    nf = nf_ref[...]
    qf = qf_ref[...]
    xi = qf[:, 0:3]
    oi = qf[:, 3:12]
    xn = nf[:, 0:3]
    on = nf[:, 3:12]
    dxn = xn - xi
    p9 = oi * jnp.concatenate([dxn, dxn, dxn], axis=1)
    du_raw = jnp.dot(p9, g9_ref[...], preferred_element_type=jnp.float32)
    dun = jnp.sqrt(jnp.sum(du_raw * du_raw, axis=1, keepdims=True))
    du = du_raw / jnp.maximum(dun, 1e-12)
    a27 = jnp.concatenate(
        [jnp.broadcast_to(oi[:, j:j + 1], (E, 3)) for j in range(9)], axis=1)
    b27 = jnp.concatenate(
        [on[:, 3 * c:3 * c + 3] for c in range(3) for _ in range(3)], axis=1)
    r9 = jnp.dot(a27 * b27, g27_ref[...], preferred_element_type=jnp.float32)
    t = jnp.dot(r9, c9_ref[...], preferred_element_type=jnp.float32)
    mag = 0.5 * jnp.sqrt(jnp.abs(1.0 + t[:, 0:3]))
    xyz = jnp.sign(t[:, 3:6]) * mag
    w = jnp.sqrt(jnp.maximum(1.0 + t[:, 6:7], 0.0)) / 2.0
    q = jnp.concatenate([xyz, w], axis=1)
    qn = jnp.sqrt(jnp.sum(q * q, axis=1, keepdims=True))
    q = q / jnp.maximum(qn, 1e-12)
    ang = dpe_ref[...] * freq_ref[...]
    rb = (dn_ref[...] - rbfmu_ref[...]) / _RBF_SIG
    rbf = jnp.exp(-(rb * rb))
    f = jnp.concatenate(
        [jnp.cos(ang), jnp.sin(ang), rbf, du, q, jnp.zeros((E, 1), jnp.float32)],
        axis=1)
    e = jnp.dot(f, wt_ref[...], preferred_element_type=jnp.float32) + b_ref[...]
    mu = jnp.mean(e, axis=1, keepdims=True)
    var = jnp.mean((e - mu) * (e - mu), axis=1, keepdims=True)
    out_ref[...] = g_ref[...] * (e - mu) / jnp.sqrt(var + 1e-6) + bt_ref[...]


def _edges(nf, qf, dpe, dn, Wt, b, gamma, beta):
    BNK = nf.shape[0]
    E = 512 if BNK % 512 == 0 else 480
    D = Wt.shape[1]
    body = functools.partial(_edge_body, E=E)
    return pl.pallas_call(
        body,
        grid=(BNK // E,),
        in_specs=[
            pl.BlockSpec((E, _FC), lambda i: (i, 0)),
            pl.BlockSpec((E, _FC), lambda i: (i, 0)),
            pl.BlockSpec((E, 1), lambda i: (i, 0)),
            pl.BlockSpec((E, 1), lambda i: (i, 0)),
            pl.BlockSpec((40, D), lambda i: (0, 0)),
            pl.BlockSpec((1, D), lambda i: (0, 0)),
            pl.BlockSpec((1, D), lambda i: (0, 0)),
            pl.BlockSpec((1, D), lambda i: (0, 0)),
            pl.BlockSpec((9, 3), lambda i: (0, 0)),
            pl.BlockSpec((27, 9), lambda i: (0, 0)),
            pl.BlockSpec((9, 7), lambda i: (0, 0)),
            pl.BlockSpec((1, NUM_PE // 2), lambda i: (0, 0)),
            pl.BlockSpec((1, NUM_RBF), lambda i: (0, 0)),
        ],
        out_specs=pl.BlockSpec((E, D), lambda i: (i, 0)),
        out_shape=jax.ShapeDtypeStruct((BNK, D), jnp.float32),
    )(nf, qf, dpe, dn, Wt, b, gamma, beta,
      jnp.asarray(_G9), jnp.asarray(_G27), jnp.asarray(_C9),
      jnp.asarray(_FREQ), jnp.asarray(_RBF_MU))


def kernel(X, mask, W, b, gamma, beta):
    B, N, _ = X.shape
    K = K_NEIGH
    D = W.shape[0]
    E_idx, Dn = _knn(X)
    O9 = _build_o9(X)
    tbl = jnp.concatenate([X, O9], axis=-1)
    offs = (jnp.arange(B, dtype=jnp.int32) * N).reshape(B, 1, 1)
    idx_g = (E_idx + offs).reshape(-1)
    nf = _sc_gather(tbl.reshape(-1), idx_g).reshape(-1, _FC)
    qf = jnp.broadcast_to(tbl[:, :, None, :], (B, N, K, _FC)).reshape(-1, _FC)
    ii = jnp.arange(N, dtype=jnp.float32).reshape(1, N, 1)
    dpe = (E_idx.astype(jnp.float32) - ii).reshape(-1, 1)
    dnf = Dn.reshape(-1, 1)
    Wt = jnp.zeros((40, D), jnp.float32).at[:W.shape[1]].set(W.T)
    E_flat = _edges(nf, qf, dpe, dnf, Wt,
                    b.reshape(1, D), gamma.reshape(1, D), beta.reshape(1, D))
    return E_flat.reshape(B, N, K, D), E_idx


# speedup vs baseline: 2.5586x; 1.1659x over previous
import functools

import numpy as np
import jax
import jax.numpy as jnp
from jax import lax
from jax.experimental import pallas as pl
from jax.experimental.pallas import tpu as pltpu
from jax.experimental.pallas import tpu_sc as plsc

K_NEIGH = 30
NUM_PE = 16
NUM_RBF = 16


def _normalize(x, axis=-1, eps=1e-12):
    n = jnp.linalg.norm(x, axis=axis, keepdims=True)
    return x / jnp.maximum(n, eps)


def _build_o9(X):
    B, N, _ = X.shape
    dX = X[:, 1:, :] - X[:, -1:, :]
    U = _normalize(dX, -1)
    u_2 = U[:, :-2, :]
    u_1 = U[:, 1:-1, :]
    n_2 = _normalize(jnp.cross(u_2, u_1), -1)
    o_1 = _normalize(u_2 - u_1, -1)
    O = jnp.stack([o_1, n_2, jnp.cross(o_1, n_2)], -1)
    O = O.reshape(B, N - 3, 9)
    return jnp.pad(O, ((0, 0), (1, 2), (0, 0)))


def _knn_body(x_ref, xt_ref, idx_ref, dn_ref, *, R, N):
    xq = x_ref[0]
    xt = xt_ref[0]
    dx = xq[:, 0:1] - xt[0:1, :]
    dy = xq[:, 1:2] - xt[1:2, :]
    dz = xq[:, 2:3] - xt[2:3, :]
    s = dx * dx + dy * dy + dz * dz
    d = jnp.sqrt(s + 1e-6)
    iota = lax.broadcasted_iota(jnp.int32, (R, N), 1)
    for j in range(K_NEIGH):
        m = jnp.min(d, axis=1, keepdims=True)
        sel = d == m
        iv = jnp.min(jnp.where(sel, iota, N), axis=1, keepdims=True)
        idx_ref[0, :, j:j + 1] = iv
        dn_ref[0, :, j:j + 1] = m
        d = jnp.where(iota == iv, jnp.float32(jnp.inf), d)


def _knn(X):
    B, N, _ = X.shape
    R = 32 if N % 32 == 0 else 8
    XT = jnp.swapaxes(X, 1, 2)
    body = functools.partial(_knn_body, R=R, N=N)
    return pl.pallas_call(
        body,
        grid=(B, N // R),
        in_specs=[
            pl.BlockSpec((1, R, 3), lambda b, n: (b, n, 0)),
            pl.BlockSpec((1, 3, N), lambda b, n: (b, 0, 0)),
        ],
        out_specs=[
            pl.BlockSpec((1, R, K_NEIGH), lambda b, n: (b, n, 0)),
            pl.BlockSpec((1, R, K_NEIGH), lambda b, n: (b, n, 0)),
        ],
        out_shape=[
            jax.ShapeDtypeStruct((B, N, K_NEIGH), jnp.int32),
            jax.ShapeDtypeStruct((B, N, K_NEIGH), jnp.float32),
        ],
    )(X, XT)


_FC = 12


def _sc_gather(tbl2, idxg):
    BNK = idxg.shape[0]
    V = tbl2.shape[0] // _FC
    NC, NS = 2, 16
    NW = NC * NS
    bpw = BNK // NW
    CH = 1920 if bpw % 1920 == 0 else bpw
    NCH = bpw // CH
    NG = CH // 16
    mesh = plsc.VectorSubcoreMesh(core_axis_name="c", subcore_axis_name="s",
                                  num_cores=NC, num_subcores=NS)

    @functools.partial(
        pl.kernel, mesh=mesh,
        out_type=jax.ShapeDtypeStruct((BNK * _FC,), jnp.float32),
        compiler_params=pltpu.CompilerParams(needs_layout_passes=False),
        scratch_types=[
            pltpu.VMEM((V * _FC,), jnp.float32),
            pltpu.VMEM((CH,), jnp.int32),
            pltpu.VMEM((CH * _FC,), jnp.float32),
        ],
    )
    def k(tbl_hbm, idx_hbm, out_hbm, tbl_v, idx_v, rows_v):
        wid = lax.axis_index("s") * NC + lax.axis_index("c")
        base = wid * bpw
        pltpu.sync_copy(tbl_hbm, tbl_v)
        iota16 = lax.iota(jnp.int32, 16)

        def chunk(c, carry):
            off = base + c * CH
            pltpu.sync_copy(idx_hbm.at[pl.ds(off, CH)], idx_v)

            def group(g, carry2):
                idx16 = idx_v[pl.ds(g * 16, 16)]
                src = idx16 * _FC
                dst = (iota16 + g * 16) * _FC
                for col in range(_FC):
                    v = plsc.load_gather(tbl_v, [src + col])
                    plsc.store_scatter(rows_v, [dst + col], v)
                return carry2

            lax.fori_loop(0, NG, group, 0)
            pltpu.sync_copy(rows_v, out_hbm.at[pl.ds(off * _FC, CH * _FC)])
            return carry

        lax.fori_loop(0, NCH, chunk, 0)

    return k(tbl2, idxg)


def _np_g(rows, cols, pairs):
    g = np.zeros((rows, cols), np.float32)
    for r, c in pairs:
        g[r, c] = 1.0
    return g

_G9 = _np_g(9, 3, [(j, j // 3) for j in range(9)])
_G27 = _np_g(27, 9, [(9 * c + 3 * a + b, 3 * a + b)
                     for c in range(3) for a in range(3) for b in range(3)])
_S9 = _np_g(3, 9, [(l % 3, l) for l in range(9)])
_S27A = _np_g(9, 27, [(3 * (l // 9) + (l % 9) // 3, l) for l in range(27)])
_S27B = _np_g(9, 27, [(3 * (l // 9) + l % 3, l) for l in range(27)])
_C9 = np.zeros((9, 7), np.float32)
_C9[0, 0], _C9[4, 0], _C9[8, 0] = 1, -1, -1
_C9[0, 1], _C9[4, 1], _C9[8, 1] = -1, 1, -1
_C9[0, 2], _C9[4, 2], _C9[8, 2] = -1, -1, 1
_C9[7, 3], _C9[5, 3] = 1, -1
_C9[2, 4], _C9[6, 4] = 1, -1
_C9[3, 5], _C9[1, 5] = 1, -1
_C9[0, 6] = _C9[4, 6] = _C9[8, 6] = 1

_FREQ = np.exp(np.arange(0, NUM_PE, 2, dtype=np.float32)
               * (-(np.log(10000.0) / NUM_PE))).reshape(1, NUM_PE // 2)
_RBF_MU = np.linspace(0.0, 20.0, NUM_RBF, dtype=np.float32).reshape(1, NUM_RBF)
_RBF_SIG = (20.0 - 0.0) / NUM_RBF


def _edge_body(nf_ref, qf_ref, dpe_ref, dn_ref, wt_ref, b_ref, g_ref, bt_ref,
               g9_ref, g27_ref, c9_ref, freq_ref, rbfmu_ref,
               s9_ref, s27a_ref, s27b_ref, out_ref, *, E):
    nf = nf_ref[...]
    qf = qf_ref[...]
    xi = qf[:, 0:3]
    oi = qf[:, 3:12]
    xn = nf[:, 0:3]
    on = nf[:, 3:12]
    dxn = xn - xi
    f32 = jnp.float32
    p9 = oi * jnp.dot(dxn, s9_ref[...], preferred_element_type=f32)
    du_raw = jnp.dot(p9, g9_ref[...], preferred_element_type=f32)
    dun = jnp.sqrt(jnp.sum(du_raw * du_raw, axis=1, keepdims=True))
    du = du_raw / jnp.maximum(dun, 1e-12)
    a27 = jnp.dot(oi, s27a_ref[...], preferred_element_type=f32)
    b27 = jnp.dot(on, s27b_ref[...], preferred_element_type=f32)
    r9 = jnp.dot(a27 * b27, g27_ref[...], preferred_element_type=f32)
    t = jnp.dot(r9, c9_ref[...], preferred_element_type=jnp.float32)
    mag = 0.5 * jnp.sqrt(jnp.abs(1.0 + t[:, 0:3]))
    xyz = jnp.sign(t[:, 3:6]) * mag
    w = jnp.sqrt(jnp.maximum(1.0 + t[:, 6:7], 0.0)) / 2.0
    q = jnp.concatenate([xyz, w], axis=1)
    qn = jnp.sqrt(jnp.sum(q * q, axis=1, keepdims=True))
    q = q / jnp.maximum(qn, 1e-12)
    ang = dpe_ref[...] * freq_ref[...]
    rb = (dn_ref[...] - rbfmu_ref[...]) / _RBF_SIG
    rbf = jnp.exp(-(rb * rb))
    f = jnp.concatenate(
        [jnp.cos(ang), jnp.sin(ang), rbf, du, q, jnp.zeros((E, 1), jnp.float32)],
        axis=1)
    e = jnp.dot(f, wt_ref[...], preferred_element_type=jnp.float32) + b_ref[...]
    mu = jnp.mean(e, axis=1, keepdims=True)
    var = jnp.mean((e - mu) * (e - mu), axis=1, keepdims=True)
    out_ref[...] = g_ref[...] * (e - mu) / jnp.sqrt(var + 1e-6) + bt_ref[...]


def _edges(nf, qf, dpe, dn, Wt, b, gamma, beta):
    BNK = nf.shape[0]
    E = 1024 if BNK % 1024 == 0 else (512 if BNK % 512 == 0 else 480)
    D = Wt.shape[1]
    body = functools.partial(_edge_body, E=E)
    return pl.pallas_call(
        body,
        grid=(BNK // E,),
        in_specs=[
            pl.BlockSpec((E, _FC), lambda i: (i, 0)),
            pl.BlockSpec((E, _FC), lambda i: (i, 0)),
            pl.BlockSpec((E, 1), lambda i: (i, 0)),
            pl.BlockSpec((E, 1), lambda i: (i, 0)),
            pl.BlockSpec((40, D), lambda i: (0, 0)),
            pl.BlockSpec((1, D), lambda i: (0, 0)),
            pl.BlockSpec((1, D), lambda i: (0, 0)),
            pl.BlockSpec((1, D), lambda i: (0, 0)),
            pl.BlockSpec((9, 3), lambda i: (0, 0)),
            pl.BlockSpec((27, 9), lambda i: (0, 0)),
            pl.BlockSpec((9, 7), lambda i: (0, 0)),
            pl.BlockSpec((1, NUM_PE // 2), lambda i: (0, 0)),
            pl.BlockSpec((1, NUM_RBF), lambda i: (0, 0)),
            pl.BlockSpec((3, 9), lambda i: (0, 0)),
            pl.BlockSpec((9, 27), lambda i: (0, 0)),
            pl.BlockSpec((9, 27), lambda i: (0, 0)),
        ],
        out_specs=pl.BlockSpec((E, D), lambda i: (i, 0)),
        out_shape=jax.ShapeDtypeStruct((BNK, D), jnp.float32),
    )(nf, qf, dpe, dn, Wt, b, gamma, beta,
      jnp.asarray(_G9), jnp.asarray(_G27), jnp.asarray(_C9),
      jnp.asarray(_FREQ), jnp.asarray(_RBF_MU),
      jnp.asarray(_S9), jnp.asarray(_S27A), jnp.asarray(_S27B))


def kernel(X, mask, W, b, gamma, beta):
    B, N, _ = X.shape
    K = K_NEIGH
    D = W.shape[0]
    E_idx, Dn = _knn(X)
    O9 = _build_o9(X)
    tbl = jnp.concatenate([X, O9], axis=-1)
    offs = (jnp.arange(B, dtype=jnp.int32) * N).reshape(B, 1, 1)
    idx_g = (E_idx + offs).reshape(-1)
    nf = _sc_gather(tbl.reshape(-1), idx_g).reshape(-1, _FC)
    qf = jnp.broadcast_to(tbl[:, :, None, :], (B, N, K, _FC)).reshape(-1, _FC)
    ii = jnp.arange(N, dtype=jnp.float32).reshape(1, N, 1)
    dpe = (E_idx.astype(jnp.float32) - ii).reshape(-1, 1)
    dnf = Dn.reshape(-1, 1)
    Wt = jnp.zeros((40, D), jnp.float32).at[:W.shape[1]].set(W.T)
    E_flat = _edges(nf, qf, dpe, dnf, Wt,
                    b.reshape(1, D), gamma.reshape(1, D), beta.reshape(1, D))
    return E_flat.reshape(B, N, K, D), E_idx


# speedup vs baseline: 3.4298x; 1.3405x over previous
import functools

import numpy as np
import jax
import jax.numpy as jnp
from jax import lax
from jax.experimental import pallas as pl
from jax.experimental.pallas import tpu as pltpu
from jax.experimental.pallas import tpu_sc as plsc

K_NEIGH = 30
NUM_PE = 16
NUM_RBF = 16


def _normalize(x, axis=-1, eps=1e-12):
    n = jnp.linalg.norm(x, axis=axis, keepdims=True)
    return x / jnp.maximum(n, eps)


def _build_o9(X):
    B, N, _ = X.shape
    dX = X[:, 1:, :] - X[:, -1:, :]
    U = _normalize(dX, -1)
    u_2 = U[:, :-2, :]
    u_1 = U[:, 1:-1, :]
    n_2 = _normalize(jnp.cross(u_2, u_1), -1)
    o_1 = _normalize(u_2 - u_1, -1)
    O = jnp.stack([o_1, n_2, jnp.cross(o_1, n_2)], -1)
    O = O.reshape(B, N - 3, 9)
    return jnp.pad(O, ((0, 0), (1, 2), (0, 0)))


def _knn_body(x_ref, xt_ref, idx_ref, dn_ref, *, R, N):
    xq = x_ref[0]
    xt = xt_ref[0]
    dx = xq[:, 0:1] - xt[0:1, :]
    dy = xq[:, 1:2] - xt[1:2, :]
    dz = xq[:, 2:3] - xt[2:3, :]
    s = dx * dx + dy * dy + dz * dz
    d = jnp.sqrt(s + 1e-6)
    iota = lax.broadcasted_iota(jnp.int32, (R, N), 1)
    for j in range(K_NEIGH):
        m = jnp.min(d, axis=1, keepdims=True)
        sel = d == m
        iv = jnp.min(jnp.where(sel, iota, N), axis=1, keepdims=True)
        idx_ref[0, :, j:j + 1] = iv
        dn_ref[0, :, j:j + 1] = m
        d = jnp.where(iota == iv, jnp.float32(jnp.inf), d)


def _knn(X):
    B, N, _ = X.shape
    R = 64 if N % 64 == 0 else 8
    XT = jnp.swapaxes(X, 1, 2)
    body = functools.partial(_knn_body, R=R, N=N)
    return pl.pallas_call(
        body,
        grid=(B, N // R),
        in_specs=[
            pl.BlockSpec((1, R, 3), lambda b, n: (b, n, 0)),
            pl.BlockSpec((1, 3, N), lambda b, n: (b, 0, 0)),
        ],
        out_specs=[
            pl.BlockSpec((1, R, K_NEIGH), lambda b, n: (b, n, 0)),
            pl.BlockSpec((1, R, K_NEIGH), lambda b, n: (b, n, 0)),
        ],
        out_shape=[
            jax.ShapeDtypeStruct((B, N, K_NEIGH), jnp.int32),
            jax.ShapeDtypeStruct((B, N, K_NEIGH), jnp.float32),
        ],
    )(X, XT)


_FC = 12


def _sc_gather(tbl2, idxg):
    BNK = idxg.shape[0]
    V = tbl2.shape[0] // _FC
    NC, NS = 2, 16
    NW = NC * NS
    bpw = BNK // NW
    CH = 1920 if bpw % 1920 == 0 else bpw
    NCH = bpw // CH
    NG = CH // 16
    mesh = plsc.VectorSubcoreMesh(core_axis_name="c", subcore_axis_name="s",
                                  num_cores=NC, num_subcores=NS)

    @functools.partial(
        pl.kernel, mesh=mesh,
        out_type=jax.ShapeDtypeStruct((BNK * _FC,), jnp.float32),
        compiler_params=pltpu.CompilerParams(needs_layout_passes=False),
        scratch_types=[
            pltpu.VMEM((V * _FC,), jnp.float32),
            pltpu.VMEM((CH,), jnp.int32),
            pltpu.VMEM((CH * _FC,), jnp.float32),
        ],
    )
    def k(tbl_hbm, idx_hbm, out_hbm, tbl_v, idx_v, rows_v):
        wid = lax.axis_index("s") * NC + lax.axis_index("c")
        base = wid * bpw
        pltpu.sync_copy(tbl_hbm, tbl_v)
        iota16 = lax.iota(jnp.int32, 16)

        def chunk(c, carry):
            off = base + c * CH
            pltpu.sync_copy(idx_hbm.at[pl.ds(off, CH)], idx_v)

            def group(g, carry2):
                idx16 = idx_v[pl.ds(g * 16, 16)]
                src = idx16 * _FC
                dst = (iota16 + g * 16) * _FC
                for col in range(_FC):
                    v = plsc.load_gather(tbl_v, [src + col])
                    plsc.store_scatter(rows_v, [dst + col], v)
                return carry2

            lax.fori_loop(0, NG, group, 0)
            pltpu.sync_copy(rows_v, out_hbm.at[pl.ds(off * _FC, CH * _FC)])
            return carry

        lax.fori_loop(0, NCH, chunk, 0)

    return k(tbl2, idxg)


def _np_g(rows, cols, pairs):
    g = np.zeros((rows, cols), np.float32)
    for r, c in pairs:
        g[r, c] = 1.0
    return g

_G9 = _np_g(9, 3, [(j, j // 3) for j in range(9)])
_G27 = _np_g(27, 9, [(9 * c + 3 * a + b, 3 * a + b)
                     for c in range(3) for a in range(3) for b in range(3)])
_S9 = _np_g(3, 9, [(l % 3, l) for l in range(9)])
_S27A = _np_g(9, 27, [(3 * (l // 9) + (l % 9) // 3, l) for l in range(27)])
_S27B = _np_g(9, 27, [(3 * (l // 9) + l % 3, l) for l in range(27)])
_C9 = np.zeros((9, 7), np.float32)
_C9[0, 0], _C9[4, 0], _C9[8, 0] = 1, -1, -1
_C9[0, 1], _C9[4, 1], _C9[8, 1] = -1, 1, -1
_C9[0, 2], _C9[4, 2], _C9[8, 2] = -1, -1, 1
_C9[7, 3], _C9[5, 3] = 1, -1
_C9[2, 4], _C9[6, 4] = 1, -1
_C9[3, 5], _C9[1, 5] = 1, -1
_C9[0, 6] = _C9[4, 6] = _C9[8, 6] = 1

_FREQ = np.exp(np.arange(0, NUM_PE, 2, dtype=np.float32)
               * (-(np.log(10000.0) / NUM_PE))).reshape(1, NUM_PE // 2)
_RBF_MU = np.linspace(0.0, 20.0, NUM_RBF, dtype=np.float32).reshape(1, NUM_RBF)
_RBF_SIG = (20.0 - 0.0) / NUM_RBF


def _edge_body(nf_ref, qf_ref, dpe_ref, dn_ref, wt_ref, b_ref, g_ref, bt_ref,
               g9_ref, g27_ref, c9_ref, freq_ref, rbfmu_ref,
               s9_ref, s27a_ref, s27b_ref, out_ref, *, E):
    nf = nf_ref[...]
    qf = qf_ref[...]
    xi = qf[:, 0:3]
    oi = qf[:, 3:12]
    xn = nf[:, 0:3]
    on = nf[:, 3:12]
    dxn = xn - xi
    f32 = jnp.float32
    p9 = oi * jnp.dot(dxn, s9_ref[...], preferred_element_type=f32)
    du_raw = jnp.dot(p9, g9_ref[...], preferred_element_type=f32)
    dun = jnp.sqrt(jnp.sum(du_raw * du_raw, axis=1, keepdims=True))
    du = du_raw / jnp.maximum(dun, 1e-12)
    a27 = jnp.dot(oi, s27a_ref[...], preferred_element_type=f32)
    b27 = jnp.dot(on, s27b_ref[...], preferred_element_type=f32)
    r9 = jnp.dot(a27 * b27, g27_ref[...], preferred_element_type=f32)
    t = jnp.dot(r9, c9_ref[...], preferred_element_type=jnp.float32)
    mag = 0.5 * jnp.sqrt(jnp.abs(1.0 + t[:, 0:3]))
    xyz = jnp.sign(t[:, 3:6]) * mag
    w = jnp.sqrt(jnp.maximum(1.0 + t[:, 6:7], 0.0)) / 2.0
    q = jnp.concatenate([xyz, w], axis=1)
    qn = jnp.sqrt(jnp.sum(q * q, axis=1, keepdims=True))
    q = q / jnp.maximum(qn, 1e-12)
    ang = dpe_ref[...] * freq_ref[...]
    rb = (dn_ref[...] - rbfmu_ref[...]) / _RBF_SIG
    rbf = jnp.exp(-(rb * rb))
    f = jnp.concatenate(
        [jnp.cos(ang), jnp.sin(ang), rbf, du, q, jnp.zeros((E, 1), jnp.float32)],
        axis=1)
    e = jnp.dot(f, wt_ref[...], preferred_element_type=jnp.float32) + b_ref[...]
    mu = jnp.mean(e, axis=1, keepdims=True)
    var = jnp.mean((e - mu) * (e - mu), axis=1, keepdims=True)
    out_ref[...] = g_ref[...] * (e - mu) / jnp.sqrt(var + 1e-6) + bt_ref[...]


def _edges(nf, qf, dpe, dn, Wt, b, gamma, beta):
    BNK = nf.shape[0]
    E = 1024 if BNK % 1024 == 0 else (512 if BNK % 512 == 0 else 480)
    D = Wt.shape[1]
    body = functools.partial(_edge_body, E=E)
    return pl.pallas_call(
        body,
        grid=(BNK // E,),
        in_specs=[
            pl.BlockSpec((E, _FC), lambda i: (i, 0)),
            pl.BlockSpec((E, _FC), lambda i: (i, 0)),
            pl.BlockSpec((E, 1), lambda i: (i, 0)),
            pl.BlockSpec((E, 1), lambda i: (i, 0)),
            pl.BlockSpec((40, D), lambda i: (0, 0)),
            pl.BlockSpec((1, D), lambda i: (0, 0)),
            pl.BlockSpec((1, D), lambda i: (0, 0)),
            pl.BlockSpec((1, D), lambda i: (0, 0)),
            pl.BlockSpec((9, 3), lambda i: (0, 0)),
            pl.BlockSpec((27, 9), lambda i: (0, 0)),
            pl.BlockSpec((9, 7), lambda i: (0, 0)),
            pl.BlockSpec((1, NUM_PE // 2), lambda i: (0, 0)),
            pl.BlockSpec((1, NUM_RBF), lambda i: (0, 0)),
            pl.BlockSpec((3, 9), lambda i: (0, 0)),
            pl.BlockSpec((9, 27), lambda i: (0, 0)),
            pl.BlockSpec((9, 27), lambda i: (0, 0)),
        ],
        out_specs=pl.BlockSpec((E, D), lambda i: (i, 0)),
        out_shape=jax.ShapeDtypeStruct((BNK, D), jnp.float32),
    )(nf, qf, dpe, dn, Wt, b, gamma, beta,
      jnp.asarray(_G9), jnp.asarray(_G27), jnp.asarray(_C9),
      jnp.asarray(_FREQ), jnp.asarray(_RBF_MU),
      jnp.asarray(_S9), jnp.asarray(_S27A), jnp.asarray(_S27B))


def kernel(X, mask, W, b, gamma, beta):
    B, N, _ = X.shape
    K = K_NEIGH
    D = W.shape[0]
    E_idx, Dn = _knn(X)
    O9 = _build_o9(X)
    tbl = jnp.concatenate([X, O9], axis=-1)
    offs = (jnp.arange(B, dtype=jnp.int32) * N).reshape(B, 1, 1)
    idx_g = (E_idx + offs).reshape(-1)
    nf = _sc_gather(tbl.reshape(-1), idx_g).reshape(-1, _FC)
    qf = jnp.broadcast_to(tbl[:, :, None, :], (B, N, K, _FC)).reshape(-1, _FC)
    ii = jnp.arange(N, dtype=jnp.float32).reshape(1, N, 1)
    dpe = (E_idx.astype(jnp.float32) - ii).reshape(-1, 1)
    dnf = Dn.reshape(-1, 1)
    Wt = jnp.zeros((40, D), jnp.float32).at[:W.shape[1]].set(W.T)
    E_flat = _edges(nf, qf, dpe, dnf, Wt,
                    b.reshape(1, D), gamma.reshape(1, D), beta.reshape(1, D))
    return E_flat.reshape(B, N, K, D), E_idx


# speedup vs baseline: 3.9554x; 1.1532x over previous
import functools

import numpy as np
import jax
import jax.numpy as jnp
from jax import lax
from jax.experimental import pallas as pl
from jax.experimental.pallas import tpu as pltpu
from jax.experimental.pallas import tpu_sc as plsc

K_NEIGH = 30
NUM_PE = 16
NUM_RBF = 16


def _normalize(x, axis=-1, eps=1e-12):
    n = jnp.linalg.norm(x, axis=axis, keepdims=True)
    return x / jnp.maximum(n, eps)


def _build_o9(X):
    B, N, _ = X.shape
    dX = X[:, 1:, :] - X[:, -1:, :]
    U = _normalize(dX, -1)
    u_2 = U[:, :-2, :]
    u_1 = U[:, 1:-1, :]
    n_2 = _normalize(jnp.cross(u_2, u_1), -1)
    o_1 = _normalize(u_2 - u_1, -1)
    O = jnp.stack([o_1, n_2, jnp.cross(o_1, n_2)], -1)
    O = O.reshape(B, N - 3, 9)
    return jnp.pad(O, ((0, 0), (1, 2), (0, 0)))


def _knn_body(x_ref, xt_ref, idx_ref, dn_ref, *, R, N):
    xq = x_ref[0]
    xt = xt_ref[0]
    dx = xq[:, 0:1] - xt[0:1, :]
    dy = xq[:, 1:2] - xt[1:2, :]
    dz = xq[:, 2:3] - xt[2:3, :]
    s = dx * dx + dy * dy + dz * dz
    d = jnp.sqrt(s + 1e-6)
    iota = lax.broadcasted_iota(jnp.int32, (R, N), 1)
    for j in range(K_NEIGH):
        m = jnp.min(d, axis=1, keepdims=True)
        sel = d == m
        iv = jnp.min(jnp.where(sel, iota, N), axis=1, keepdims=True)
        idx_ref[0, :, j:j + 1] = iv
        dn_ref[0, :, j:j + 1] = m
        d = jnp.where(iota == iv, jnp.float32(jnp.inf), d)


def _knn(X):
    B, N, _ = X.shape
    R = 128 if N % 128 == 0 else 8
    XT = jnp.swapaxes(X, 1, 2)
    body = functools.partial(_knn_body, R=R, N=N)
    return pl.pallas_call(
        body,
        grid=(B, N // R),
        in_specs=[
            pl.BlockSpec((1, R, 3), lambda b, n: (b, n, 0)),
            pl.BlockSpec((1, 3, N), lambda b, n: (b, 0, 0)),
        ],
        out_specs=[
            pl.BlockSpec((1, R, K_NEIGH), lambda b, n: (b, n, 0)),
            pl.BlockSpec((1, R, K_NEIGH), lambda b, n: (b, n, 0)),
        ],
        out_shape=[
            jax.ShapeDtypeStruct((B, N, K_NEIGH), jnp.int32),
            jax.ShapeDtypeStruct((B, N, K_NEIGH), jnp.float32),
        ],
    )(X, XT)


_FC = 12


def _sc_gather(tbl2, idxg):
    BNK = idxg.shape[0]
    V = tbl2.shape[0] // _FC
    NC, NS = 2, 16
    NW = NC * NS
    bpw = BNK // NW
    CH = 1920 if bpw % 1920 == 0 else bpw
    NCH = bpw // CH
    NG = CH // 16
    mesh = plsc.VectorSubcoreMesh(core_axis_name="c", subcore_axis_name="s",
                                  num_cores=NC, num_subcores=NS)

    @functools.partial(
        pl.kernel, mesh=mesh,
        out_type=jax.ShapeDtypeStruct((BNK * _FC,), jnp.float32),
        compiler_params=pltpu.CompilerParams(needs_layout_passes=False),
        scratch_types=[
            pltpu.VMEM((V * _FC,), jnp.float32),
            pltpu.VMEM((CH,), jnp.int32),
            pltpu.VMEM((CH * _FC,), jnp.float32),
        ],
    )
    def k(tbl_hbm, idx_hbm, out_hbm, tbl_v, idx_v, rows_v):
        wid = lax.axis_index("s") * NC + lax.axis_index("c")
        base = wid * bpw
        pltpu.sync_copy(tbl_hbm, tbl_v)
        iota16 = lax.iota(jnp.int32, 16)

        def chunk(c, carry):
            off = base + c * CH
            pltpu.sync_copy(idx_hbm.at[pl.ds(off, CH)], idx_v)

            def group(g, carry2):
                idx16 = idx_v[pl.ds(g * 16, 16)]
                src = idx16 * _FC
                dst = (iota16 + g * 16) * _FC
                for col in range(_FC):
                    v = plsc.load_gather(tbl_v, [src + col])
                    plsc.store_scatter(rows_v, [dst + col], v)
                return carry2

            lax.fori_loop(0, NG, group, 0)
            pltpu.sync_copy(rows_v, out_hbm.at[pl.ds(off * _FC, CH * _FC)])
            return carry

        lax.fori_loop(0, NCH, chunk, 0)

    return k(tbl2, idxg)


def _np_g(rows, cols, pairs):
    g = np.zeros((rows, cols), np.float32)
    for r, c in pairs:
        g[r, c] = 1.0
    return g

_G9 = _np_g(9, 3, [(j, j // 3) for j in range(9)])
_G27 = _np_g(27, 9, [(9 * c + 3 * a + b, 3 * a + b)
                     for c in range(3) for a in range(3) for b in range(3)])
_S9 = _np_g(3, 9, [(l % 3, l) for l in range(9)])
_S27A = _np_g(9, 27, [(3 * (l // 9) + (l % 9) // 3, l) for l in range(27)])
_S27B = _np_g(9, 27, [(3 * (l // 9) + l % 3, l) for l in range(27)])
_C9 = np.zeros((9, 7), np.float32)
_C9[0, 0], _C9[4, 0], _C9[8, 0] = 1, -1, -1
_C9[0, 1], _C9[4, 1], _C9[8, 1] = -1, 1, -1
_C9[0, 2], _C9[4, 2], _C9[8, 2] = -1, -1, 1
_C9[7, 3], _C9[5, 3] = 1, -1
_C9[2, 4], _C9[6, 4] = 1, -1
_C9[3, 5], _C9[1, 5] = 1, -1
_C9[0, 6] = _C9[4, 6] = _C9[8, 6] = 1

_FREQ = np.exp(np.arange(0, NUM_PE, 2, dtype=np.float32)
               * (-(np.log(10000.0) / NUM_PE))).reshape(1, NUM_PE // 2)
_RBF_MU = np.linspace(0.0, 20.0, NUM_RBF, dtype=np.float32).reshape(1, NUM_RBF)
_RBF_SIG = (20.0 - 0.0) / NUM_RBF


def _edge_body(nf_ref, qf_ref, dpe_ref, dn_ref, wt_ref, b_ref, g_ref, bt_ref,
               g9_ref, g27_ref, c9_ref, freq_ref, rbfmu_ref,
               s9_ref, s27a_ref, s27b_ref, out_ref, *, E):
    nf = nf_ref[...]
    qf = qf_ref[...]
    xi = qf[:, 0:3]
    oi = qf[:, 3:12]
    xn = nf[:, 0:3]
    on = nf[:, 3:12]
    dxn = xn - xi
    f32 = jnp.float32
    p9 = oi * jnp.dot(dxn, s9_ref[...], preferred_element_type=f32)
    du_raw = jnp.dot(p9, g9_ref[...], preferred_element_type=f32)
    dun = jnp.sqrt(jnp.sum(du_raw * du_raw, axis=1, keepdims=True))
    du = du_raw / jnp.maximum(dun, 1e-12)
    a27 = jnp.dot(oi, s27a_ref[...], preferred_element_type=f32)
    b27 = jnp.dot(on, s27b_ref[...], preferred_element_type=f32)
    r9 = jnp.dot(a27 * b27, g27_ref[...], preferred_element_type=f32)
    t = jnp.dot(r9, c9_ref[...], preferred_element_type=jnp.float32)
    mag = 0.5 * jnp.sqrt(jnp.abs(1.0 + t[:, 0:3]))
    xyz = jnp.sign(t[:, 3:6]) * mag
    w = jnp.sqrt(jnp.maximum(1.0 + t[:, 6:7], 0.0)) / 2.0
    q = jnp.concatenate([xyz, w], axis=1)
    qn = jnp.sqrt(jnp.sum(q * q, axis=1, keepdims=True))
    q = q / jnp.maximum(qn, 1e-12)
    ang = dpe_ref[...] * freq_ref[...]
    rb = (dn_ref[...] - rbfmu_ref[...]) / _RBF_SIG
    rbf = jnp.exp(-(rb * rb))
    f = jnp.concatenate(
        [jnp.cos(ang), jnp.sin(ang), rbf, du, q, jnp.zeros((E, 1), jnp.float32)],
        axis=1)
    e = jnp.dot(f, wt_ref[...], preferred_element_type=jnp.float32) + b_ref[...]
    mu = jnp.mean(e, axis=1, keepdims=True)
    var = jnp.mean((e - mu) * (e - mu), axis=1, keepdims=True)
    out_ref[...] = g_ref[...] * (e - mu) / jnp.sqrt(var + 1e-6) + bt_ref[...]


def _edges(nf, qf, dpe, dn, Wt, b, gamma, beta):
    BNK = nf.shape[0]
    E = 1024 if BNK % 1024 == 0 else (512 if BNK % 512 == 0 else 480)
    D = Wt.shape[1]
    body = functools.partial(_edge_body, E=E)
    return pl.pallas_call(
        body,
        grid=(BNK // E,),
        in_specs=[
            pl.BlockSpec((E, _FC), lambda i: (i, 0)),
            pl.BlockSpec((E, _FC), lambda i: (i, 0)),
            pl.BlockSpec((E, 1), lambda i: (i, 0)),
            pl.BlockSpec((E, 1), lambda i: (i, 0)),
            pl.BlockSpec((40, D), lambda i: (0, 0)),
            pl.BlockSpec((1, D), lambda i: (0, 0)),
            pl.BlockSpec((1, D), lambda i: (0, 0)),
            pl.BlockSpec((1, D), lambda i: (0, 0)),
            pl.BlockSpec((9, 3), lambda i: (0, 0)),
            pl.BlockSpec((27, 9), lambda i: (0, 0)),
            pl.BlockSpec((9, 7), lambda i: (0, 0)),
            pl.BlockSpec((1, NUM_PE // 2), lambda i: (0, 0)),
            pl.BlockSpec((1, NUM_RBF), lambda i: (0, 0)),
            pl.BlockSpec((3, 9), lambda i: (0, 0)),
            pl.BlockSpec((9, 27), lambda i: (0, 0)),
            pl.BlockSpec((9, 27), lambda i: (0, 0)),
        ],
        out_specs=pl.BlockSpec((E, D), lambda i: (i, 0)),
        out_shape=jax.ShapeDtypeStruct((BNK, D), jnp.float32),
    )(nf, qf, dpe, dn, Wt, b, gamma, beta,
      jnp.asarray(_G9), jnp.asarray(_G27), jnp.asarray(_C9),
      jnp.asarray(_FREQ), jnp.asarray(_RBF_MU),
      jnp.asarray(_S9), jnp.asarray(_S27A), jnp.asarray(_S27B))


def kernel(X, mask, W, b, gamma, beta):
    B, N, _ = X.shape
    K = K_NEIGH
    D = W.shape[0]
    E_idx, Dn = _knn(X)
    O9 = _build_o9(X)
    tbl = jnp.concatenate([X, O9], axis=-1)
    offs = (jnp.arange(B, dtype=jnp.int32) * N).reshape(B, 1, 1)
    idx_g = (E_idx + offs).reshape(-1)
    nf = _sc_gather(tbl.reshape(-1), idx_g).reshape(-1, _FC)
    qf = jnp.broadcast_to(tbl[:, :, None, :], (B, N, K, _FC)).reshape(-1, _FC)
    ii = jnp.arange(N, dtype=jnp.float32).reshape(1, N, 1)
    dpe = (E_idx.astype(jnp.float32) - ii).reshape(-1, 1)
    dnf = Dn.reshape(-1, 1)
    Wt = jnp.zeros((40, D), jnp.float32).at[:W.shape[1]].set(W.T)
    E_flat = _edges(nf, qf, dpe, dnf, Wt,
                    b.reshape(1, D), gamma.reshape(1, D), beta.reshape(1, D))
    return E_flat.reshape(B, N, K, D), E_idx


# speedup vs baseline: 4.0965x; 1.0357x over previous
import functools

import numpy as np
import jax
import jax.numpy as jnp
from jax import lax
from jax.experimental import pallas as pl
from jax.experimental.pallas import tpu as pltpu
from jax.experimental.pallas import tpu_sc as plsc

K_NEIGH = 30
NUM_PE = 16
NUM_RBF = 16


def _normalize(x, axis=-1, eps=1e-12):
    n = jnp.linalg.norm(x, axis=axis, keepdims=True)
    return x / jnp.maximum(n, eps)


def _build_o9(X):
    B, N, _ = X.shape
    dX = X[:, 1:, :] - X[:, -1:, :]
    U = _normalize(dX, -1)
    u_2 = U[:, :-2, :]
    u_1 = U[:, 1:-1, :]
    n_2 = _normalize(jnp.cross(u_2, u_1), -1)
    o_1 = _normalize(u_2 - u_1, -1)
    O = jnp.stack([o_1, n_2, jnp.cross(o_1, n_2)], -1)
    O = O.reshape(B, N - 3, 9)
    return jnp.pad(O, ((0, 0), (1, 2), (0, 0)))


def _knn_body(x_ref, xt_ref, idx_ref, dn_ref, *, R, N):
    xq = x_ref[0]
    xt = xt_ref[0]
    dx = xq[:, 0:1] - xt[0:1, :]
    dy = xq[:, 1:2] - xt[1:2, :]
    dz = xq[:, 2:3] - xt[2:3, :]
    s = dx * dx + dy * dy + dz * dz
    d = jnp.sqrt(s + 1e-6)
    iota = lax.broadcasted_iota(jnp.int32, (R, N), 1)
    for j in range(K_NEIGH):
        m = jnp.min(d, axis=1, keepdims=True)
        sel = d == m
        iv = jnp.min(jnp.where(sel, iota, N), axis=1, keepdims=True)
        idx_ref[0, :, j:j + 1] = iv
        dn_ref[0, :, j:j + 1] = m
        d = jnp.where(iota == iv, jnp.float32(jnp.inf), d)


def _knn(X):
    B, N, _ = X.shape
    R = 256 if N % 256 == 0 else 8
    XT = jnp.swapaxes(X, 1, 2)
    body = functools.partial(_knn_body, R=R, N=N)
    return pl.pallas_call(
        body,
        grid=(B, N // R),
        in_specs=[
            pl.BlockSpec((1, R, 3), lambda b, n: (b, n, 0)),
            pl.BlockSpec((1, 3, N), lambda b, n: (b, 0, 0)),
        ],
        out_specs=[
            pl.BlockSpec((1, R, K_NEIGH), lambda b, n: (b, n, 0)),
            pl.BlockSpec((1, R, K_NEIGH), lambda b, n: (b, n, 0)),
        ],
        out_shape=[
            jax.ShapeDtypeStruct((B, N, K_NEIGH), jnp.int32),
            jax.ShapeDtypeStruct((B, N, K_NEIGH), jnp.float32),
        ],
    )(X, XT)


_FC = 12


def _sc_gather(tbl2, idxg):
    BNK = idxg.shape[0]
    V = tbl2.shape[0] // _FC
    NC, NS = 2, 16
    NW = NC * NS
    bpw = BNK // NW
    CH = 1920 if bpw % 1920 == 0 else bpw
    NCH = bpw // CH
    NG = CH // 16
    mesh = plsc.VectorSubcoreMesh(core_axis_name="c", subcore_axis_name="s",
                                  num_cores=NC, num_subcores=NS)

    @functools.partial(
        pl.kernel, mesh=mesh,
        out_type=jax.ShapeDtypeStruct((BNK * _FC,), jnp.float32),
        compiler_params=pltpu.CompilerParams(needs_layout_passes=False),
        scratch_types=[
            pltpu.VMEM((V * _FC,), jnp.float32),
            pltpu.VMEM((CH,), jnp.int32),
            pltpu.VMEM((CH * _FC,), jnp.float32),
        ],
    )
    def k(tbl_hbm, idx_hbm, out_hbm, tbl_v, idx_v, rows_v):
        wid = lax.axis_index("s") * NC + lax.axis_index("c")
        base = wid * bpw
        pltpu.sync_copy(tbl_hbm, tbl_v)
        iota16 = lax.iota(jnp.int32, 16)

        def chunk(c, carry):
            off = base + c * CH
            pltpu.sync_copy(idx_hbm.at[pl.ds(off, CH)], idx_v)

            def group(g, carry2):
                idx16 = idx_v[pl.ds(g * 16, 16)]
                src = idx16 * _FC
                dst = (iota16 + g * 16) * _FC
                for col in range(_FC):
                    v = plsc.load_gather(tbl_v, [src + col])
                    plsc.store_scatter(rows_v, [dst + col], v)
                return carry2

            lax.fori_loop(0, NG, group, 0)
            pltpu.sync_copy(rows_v, out_hbm.at[pl.ds(off * _FC, CH * _FC)])
            return carry

        lax.fori_loop(0, NCH, chunk, 0)

    return k(tbl2, idxg)


def _np_g(rows, cols, pairs):
    g = np.zeros((rows, cols), np.float32)
    for r, c in pairs:
        g[r, c] = 1.0
    return g

_G9 = _np_g(9, 3, [(j, j // 3) for j in range(9)])
_G27 = _np_g(27, 9, [(9 * c + 3 * a + b, 3 * a + b)
                     for c in range(3) for a in range(3) for b in range(3)])
_S9 = _np_g(3, 9, [(l % 3, l) for l in range(9)])
_S27A = _np_g(9, 27, [(3 * (l // 9) + (l % 9) // 3, l) for l in range(27)])
_S27B = _np_g(9, 27, [(3 * (l // 9) + l % 3, l) for l in range(27)])
_C9 = np.zeros((9, 7), np.float32)
_C9[0, 0], _C9[4, 0], _C9[8, 0] = 1, -1, -1
_C9[0, 1], _C9[4, 1], _C9[8, 1] = -1, 1, -1
_C9[0, 2], _C9[4, 2], _C9[8, 2] = -1, -1, 1
_C9[7, 3], _C9[5, 3] = 1, -1
_C9[2, 4], _C9[6, 4] = 1, -1
_C9[3, 5], _C9[1, 5] = 1, -1
_C9[0, 6] = _C9[4, 6] = _C9[8, 6] = 1

_FREQ = np.exp(np.arange(0, NUM_PE, 2, dtype=np.float32)
               * (-(np.log(10000.0) / NUM_PE))).reshape(1, NUM_PE // 2)
_RBF_MU = np.linspace(0.0, 20.0, NUM_RBF, dtype=np.float32).reshape(1, NUM_RBF)
_RBF_SIG = (20.0 - 0.0) / NUM_RBF


def _edge_body(nf_ref, qf_ref, dpe_ref, dn_ref, wt_ref, b_ref, g_ref, bt_ref,
               g9_ref, g27_ref, c9_ref, freq_ref, rbfmu_ref,
               s9_ref, s27a_ref, s27b_ref, out_ref, *, E):
    nf = nf_ref[...]
    qf = qf_ref[...]
    xi = qf[:, 0:3]
    oi = qf[:, 3:12]
    xn = nf[:, 0:3]
    on = nf[:, 3:12]
    dxn = xn - xi
    f32 = jnp.float32
    p9 = oi * jnp.dot(dxn, s9_ref[...], preferred_element_type=f32)
    du_raw = jnp.dot(p9, g9_ref[...], preferred_element_type=f32)
    dun = jnp.sqrt(jnp.sum(du_raw * du_raw, axis=1, keepdims=True))
    du = du_raw / jnp.maximum(dun, 1e-12)
    a27 = jnp.dot(oi, s27a_ref[...], preferred_element_type=f32)
    b27 = jnp.dot(on, s27b_ref[...], preferred_element_type=f32)
    r9 = jnp.dot(a27 * b27, g27_ref[...], preferred_element_type=f32)
    t = jnp.dot(r9, c9_ref[...], preferred_element_type=jnp.float32)
    mag = 0.5 * jnp.sqrt(jnp.abs(1.0 + t[:, 0:3]))
    xyz = jnp.sign(t[:, 3:6]) * mag
    w = jnp.sqrt(jnp.maximum(1.0 + t[:, 6:7], 0.0)) / 2.0
    q = jnp.concatenate([xyz, w], axis=1)
    qn = jnp.sqrt(jnp.sum(q * q, axis=1, keepdims=True))
    q = q / jnp.maximum(qn, 1e-12)
    ang = dpe_ref[...] * freq_ref[...]
    rb = (dn_ref[...] - rbfmu_ref[...]) / _RBF_SIG
    rbf = jnp.exp(-(rb * rb))
    f = jnp.concatenate(
        [jnp.cos(ang), jnp.sin(ang), rbf, du, q, jnp.zeros((E, 1), jnp.float32)],
        axis=1)
    e = jnp.dot(f, wt_ref[...], preferred_element_type=jnp.float32) + b_ref[...]
    mu = jnp.mean(e, axis=1, keepdims=True)
    var = jnp.mean((e - mu) * (e - mu), axis=1, keepdims=True)
    out_ref[...] = g_ref[...] * (e - mu) / jnp.sqrt(var + 1e-6) + bt_ref[...]


def _edges(nf, qf, dpe, dn, Wt, b, gamma, beta):
    BNK = nf.shape[0]
    E = 1024 if BNK % 1024 == 0 else (512 if BNK % 512 == 0 else 480)
    D = Wt.shape[1]
    body = functools.partial(_edge_body, E=E)
    return pl.pallas_call(
        body,
        grid=(BNK // E,),
        in_specs=[
            pl.BlockSpec((E, _FC), lambda i: (i, 0)),
            pl.BlockSpec((E, _FC), lambda i: (i, 0)),
            pl.BlockSpec((E, 1), lambda i: (i, 0)),
            pl.BlockSpec((E, 1), lambda i: (i, 0)),
            pl.BlockSpec((40, D), lambda i: (0, 0)),
            pl.BlockSpec((1, D), lambda i: (0, 0)),
            pl.BlockSpec((1, D), lambda i: (0, 0)),
            pl.BlockSpec((1, D), lambda i: (0, 0)),
            pl.BlockSpec((9, 3), lambda i: (0, 0)),
            pl.BlockSpec((27, 9), lambda i: (0, 0)),
            pl.BlockSpec((9, 7), lambda i: (0, 0)),
            pl.BlockSpec((1, NUM_PE // 2), lambda i: (0, 0)),
            pl.BlockSpec((1, NUM_RBF), lambda i: (0, 0)),
            pl.BlockSpec((3, 9), lambda i: (0, 0)),
            pl.BlockSpec((9, 27), lambda i: (0, 0)),
            pl.BlockSpec((9, 27), lambda i: (0, 0)),
        ],
        out_specs=pl.BlockSpec((E, D), lambda i: (i, 0)),
        out_shape=jax.ShapeDtypeStruct((BNK, D), jnp.float32),
    )(nf, qf, dpe, dn, Wt, b, gamma, beta,
      jnp.asarray(_G9), jnp.asarray(_G27), jnp.asarray(_C9),
      jnp.asarray(_FREQ), jnp.asarray(_RBF_MU),
      jnp.asarray(_S9), jnp.asarray(_S27A), jnp.asarray(_S27B))


def kernel(X, mask, W, b, gamma, beta):
    B, N, _ = X.shape
    K = K_NEIGH
    D = W.shape[0]
    E_idx, Dn = _knn(X)
    O9 = _build_o9(X)
    tbl = jnp.concatenate([X, O9], axis=-1)
    offs = (jnp.arange(B, dtype=jnp.int32) * N).reshape(B, 1, 1)
    idx_g = (E_idx + offs).reshape(-1)
    nf = _sc_gather(tbl.reshape(-1), idx_g).reshape(-1, _FC)
    qf = jnp.broadcast_to(tbl[:, :, None, :], (B, N, K, _FC)).reshape(-1, _FC)
    ii = jnp.arange(N, dtype=jnp.float32).reshape(1, N, 1)
    dpe = (E_idx.astype(jnp.float32) - ii).reshape(-1, 1)
    dnf = Dn.reshape(-1, 1)
    Wt = jnp.zeros((40, D), jnp.float32).at[:W.shape[1]].set(W.T)
    E_flat = _edges(nf, qf, dpe, dnf, Wt,
                    b.reshape(1, D), gamma.reshape(1, D), beta.reshape(1, D))
    return E_flat.reshape(B, N, K, D), E_idx


# speedup vs baseline: 4.1404x; 1.0107x over previous
import functools

import numpy as np
import jax
import jax.numpy as jnp
from jax import lax
from jax.experimental import pallas as pl
from jax.experimental.pallas import tpu as pltpu
from jax.experimental.pallas import tpu_sc as plsc

K_NEIGH = 30
NUM_PE = 16
NUM_RBF = 16


def _normalize(x, axis=-1, eps=1e-12):
    n = jnp.linalg.norm(x, axis=axis, keepdims=True)
    return x / jnp.maximum(n, eps)


def _build_o9(X):
    B, N, _ = X.shape
    dX = X[:, 1:, :] - X[:, -1:, :]
    U = _normalize(dX, -1)
    u_2 = U[:, :-2, :]
    u_1 = U[:, 1:-1, :]
    n_2 = _normalize(jnp.cross(u_2, u_1), -1)
    o_1 = _normalize(u_2 - u_1, -1)
    O = jnp.stack([o_1, n_2, jnp.cross(o_1, n_2)], -1)
    O = O.reshape(B, N - 3, 9)
    return jnp.pad(O, ((0, 0), (1, 2), (0, 0)))


def _knn_body(x_ref, xt_ref, idx_ref, dn_ref, *, R, N):
    xq = x_ref[0]
    xt = xt_ref[0]
    dx = xq[:, 0:1] - xt[0:1, :]
    dy = xq[:, 1:2] - xt[1:2, :]
    dz = xq[:, 2:3] - xt[2:3, :]
    s = dx * dx + dy * dy + dz * dz
    d = jnp.sqrt(s + 1e-6)
    iota = lax.broadcasted_iota(jnp.int32, (R, N), 1)
    for j in range(K_NEIGH):
        m = jnp.min(d, axis=1, keepdims=True)
        sel = d == m
        iv = jnp.min(jnp.where(sel, iota, N), axis=1, keepdims=True)
        idx_ref[0, :, j:j + 1] = iv
        dn_ref[0, :, j:j + 1] = m
        d = jnp.where(iota == iv, jnp.float32(jnp.inf), d)


def _knn(X):
    B, N, _ = X.shape
    R = 512 if N % 512 == 0 else 8
    XT = jnp.swapaxes(X, 1, 2)
    body = functools.partial(_knn_body, R=R, N=N)
    return pl.pallas_call(
        body,
        grid=(B, N // R),
        in_specs=[
            pl.BlockSpec((1, R, 3), lambda b, n: (b, n, 0)),
            pl.BlockSpec((1, 3, N), lambda b, n: (b, 0, 0)),
        ],
        out_specs=[
            pl.BlockSpec((1, R, K_NEIGH), lambda b, n: (b, n, 0)),
            pl.BlockSpec((1, R, K_NEIGH), lambda b, n: (b, n, 0)),
        ],
        out_shape=[
            jax.ShapeDtypeStruct((B, N, K_NEIGH), jnp.int32),
            jax.ShapeDtypeStruct((B, N, K_NEIGH), jnp.float32),
        ],
    )(X, XT)


_FC = 12


def _sc_gather(tbl2, idxg):
    BNK = idxg.shape[0]
    V = tbl2.shape[0] // _FC
    NC, NS = 2, 16
    NW = NC * NS
    bpw = BNK // NW
    CH = 1920 if bpw % 1920 == 0 else bpw
    NCH = bpw // CH
    NG = CH // 16
    mesh = plsc.VectorSubcoreMesh(core_axis_name="c", subcore_axis_name="s",
                                  num_cores=NC, num_subcores=NS)

    @functools.partial(
        pl.kernel, mesh=mesh,
        out_type=jax.ShapeDtypeStruct((BNK * _FC,), jnp.float32),
        compiler_params=pltpu.CompilerParams(needs_layout_passes=False),
        scratch_types=[
            pltpu.VMEM((V * _FC,), jnp.float32),
            pltpu.VMEM((CH,), jnp.int32),
            pltpu.VMEM((CH * _FC,), jnp.float32),
        ],
    )
    def k(tbl_hbm, idx_hbm, out_hbm, tbl_v, idx_v, rows_v):
        wid = lax.axis_index("s") * NC + lax.axis_index("c")
        base = wid * bpw
        pltpu.sync_copy(tbl_hbm, tbl_v)
        iota16 = lax.iota(jnp.int32, 16)

        def chunk(c, carry):
            off = base + c * CH
            pltpu.sync_copy(idx_hbm.at[pl.ds(off, CH)], idx_v)

            def group(g, carry2):
                idx16 = idx_v[pl.ds(g * 16, 16)]
                src = idx16 * _FC
                dst = (iota16 + g * 16) * _FC
                for col in range(_FC):
                    v = plsc.load_gather(tbl_v, [src + col])
                    plsc.store_scatter(rows_v, [dst + col], v)
                return carry2

            lax.fori_loop(0, NG, group, 0)
            pltpu.sync_copy(rows_v, out_hbm.at[pl.ds(off * _FC, CH * _FC)])
            return carry

        lax.fori_loop(0, NCH, chunk, 0)

    return k(tbl2, idxg)


def _np_g(rows, cols, pairs):
    g = np.zeros((rows, cols), np.float32)
    for r, c in pairs:
        g[r, c] = 1.0
    return g

_G9 = _np_g(9, 3, [(j, j // 3) for j in range(9)])
_G27 = _np_g(27, 9, [(9 * c + 3 * a + b, 3 * a + b)
                     for c in range(3) for a in range(3) for b in range(3)])
_S9 = _np_g(3, 9, [(l % 3, l) for l in range(9)])
_S27A = _np_g(9, 27, [(3 * (l // 9) + (l % 9) // 3, l) for l in range(27)])
_S27B = _np_g(9, 27, [(3 * (l // 9) + l % 3, l) for l in range(27)])
_C9 = np.zeros((9, 7), np.float32)
_C9[0, 0], _C9[4, 0], _C9[8, 0] = 1, -1, -1
_C9[0, 1], _C9[4, 1], _C9[8, 1] = -1, 1, -1
_C9[0, 2], _C9[4, 2], _C9[8, 2] = -1, -1, 1
_C9[7, 3], _C9[5, 3] = 1, -1
_C9[2, 4], _C9[6, 4] = 1, -1
_C9[3, 5], _C9[1, 5] = 1, -1
_C9[0, 6] = _C9[4, 6] = _C9[8, 6] = 1

_FREQ = np.exp(np.arange(0, NUM_PE, 2, dtype=np.float32)
               * (-(np.log(10000.0) / NUM_PE))).reshape(1, NUM_PE // 2)
_RBF_MU = np.linspace(0.0, 20.0, NUM_RBF, dtype=np.float32).reshape(1, NUM_RBF)
_RBF_SIG = (20.0 - 0.0) / NUM_RBF


def _edge_body(nf_ref, qf_ref, dpe_ref, dn_ref, wt_ref, b_ref, g_ref, bt_ref,
               g9_ref, g27_ref, c9_ref, freq_ref, rbfmu_ref,
               s9_ref, s27a_ref, s27b_ref, out_ref, *, E):
    nf = nf_ref[...]
    qf = qf_ref[...]
    xi = qf[:, 0:3]
    oi = qf[:, 3:12]
    xn = nf[:, 0:3]
    on = nf[:, 3:12]
    dxn = xn - xi
    f32 = jnp.float32
    p9 = oi * jnp.dot(dxn, s9_ref[...], preferred_element_type=f32)
    du_raw = jnp.dot(p9, g9_ref[...], preferred_element_type=f32)
    dun = jnp.sqrt(jnp.sum(du_raw * du_raw, axis=1, keepdims=True))
    du = du_raw / jnp.maximum(dun, 1e-12)
    a27 = jnp.dot(oi, s27a_ref[...], preferred_element_type=f32)
    b27 = jnp.dot(on, s27b_ref[...], preferred_element_type=f32)
    r9 = jnp.dot(a27 * b27, g27_ref[...], preferred_element_type=f32)
    t = jnp.dot(r9, c9_ref[...], preferred_element_type=jnp.float32)
    mag = 0.5 * jnp.sqrt(jnp.abs(1.0 + t[:, 0:3]))
    xyz = jnp.sign(t[:, 3:6]) * mag
    w = jnp.sqrt(jnp.maximum(1.0 + t[:, 6:7], 0.0)) / 2.0
    q = jnp.concatenate([xyz, w], axis=1)
    qn = jnp.sqrt(jnp.sum(q * q, axis=1, keepdims=True))
    q = q / jnp.maximum(qn, 1e-12)
    ang = dpe_ref[...] * freq_ref[...]
    rb = (dn_ref[...] - rbfmu_ref[...]) / _RBF_SIG
    rbf = jnp.exp(-(rb * rb))
    f = jnp.concatenate(
        [jnp.cos(ang), jnp.sin(ang), rbf, du, q, jnp.zeros((E, 1), jnp.float32)],
        axis=1)
    e = jnp.dot(f, wt_ref[...], preferred_element_type=jnp.float32) + b_ref[...]
    mu = jnp.mean(e, axis=1, keepdims=True)
    var = jnp.mean((e - mu) * (e - mu), axis=1, keepdims=True)
    out_ref[...] = g_ref[...] * (e - mu) / jnp.sqrt(var + 1e-6) + bt_ref[...]


def _edges(nf, qf, dpe, dn, Wt, b, gamma, beta):
    BNK = nf.shape[0]
    E = 1024 if BNK % 1024 == 0 else (512 if BNK % 512 == 0 else 480)
    D = Wt.shape[1]
    body = functools.partial(_edge_body, E=E)
    return pl.pallas_call(
        body,
        grid=(BNK // E,),
        in_specs=[
            pl.BlockSpec((E, _FC), lambda i: (i, 0)),
            pl.BlockSpec((E, _FC), lambda i: (i, 0)),
            pl.BlockSpec((E, 1), lambda i: (i, 0)),
            pl.BlockSpec((E, 1), lambda i: (i, 0)),
            pl.BlockSpec((40, D), lambda i: (0, 0)),
            pl.BlockSpec((1, D), lambda i: (0, 0)),
            pl.BlockSpec((1, D), lambda i: (0, 0)),
            pl.BlockSpec((1, D), lambda i: (0, 0)),
            pl.BlockSpec((9, 3), lambda i: (0, 0)),
            pl.BlockSpec((27, 9), lambda i: (0, 0)),
            pl.BlockSpec((9, 7), lambda i: (0, 0)),
            pl.BlockSpec((1, NUM_PE // 2), lambda i: (0, 0)),
            pl.BlockSpec((1, NUM_RBF), lambda i: (0, 0)),
            pl.BlockSpec((3, 9), lambda i: (0, 0)),
            pl.BlockSpec((9, 27), lambda i: (0, 0)),
            pl.BlockSpec((9, 27), lambda i: (0, 0)),
        ],
        out_specs=pl.BlockSpec((E, D), lambda i: (i, 0)),
        out_shape=jax.ShapeDtypeStruct((BNK, D), jnp.float32),
    )(nf, qf, dpe, dn, Wt, b, gamma, beta,
      jnp.asarray(_G9), jnp.asarray(_G27), jnp.asarray(_C9),
      jnp.asarray(_FREQ), jnp.asarray(_RBF_MU),
      jnp.asarray(_S9), jnp.asarray(_S27A), jnp.asarray(_S27B))


def kernel(X, mask, W, b, gamma, beta):
    B, N, _ = X.shape
    K = K_NEIGH
    D = W.shape[0]
    E_idx, Dn = _knn(X)
    O9 = _build_o9(X)
    tbl = jnp.concatenate([X, O9], axis=-1)
    offs = (jnp.arange(B, dtype=jnp.int32) * N).reshape(B, 1, 1)
    idx_g = (E_idx + offs).reshape(-1)
    nf = _sc_gather(tbl.reshape(-1), idx_g).reshape(-1, _FC)
    qf = jnp.broadcast_to(tbl[:, :, None, :], (B, N, K, _FC)).reshape(-1, _FC)
    ii = jnp.arange(N, dtype=jnp.float32).reshape(1, N, 1)
    dpe = (E_idx.astype(jnp.float32) - ii).reshape(-1, 1)
    dnf = Dn.reshape(-1, 1)
    Wt = jnp.zeros((40, D), jnp.float32).at[:W.shape[1]].set(W.T)
    E_flat = _edges(nf, qf, dpe, dnf, Wt,
                    b.reshape(1, D), gamma.reshape(1, D), beta.reshape(1, D))
    return E_flat.reshape(B, N, K, D), E_idx


# speedup vs baseline: 4.3155x; 1.0423x over previous
import functools

import numpy as np
import jax
import jax.numpy as jnp
from jax import lax
from jax.experimental import pallas as pl
from jax.experimental.pallas import tpu as pltpu
from jax.experimental.pallas import tpu_sc as plsc

K_NEIGH = 30
NUM_PE = 16
NUM_RBF = 16


def _normalize(x, axis=-1, eps=1e-12):
    n = jnp.linalg.norm(x, axis=axis, keepdims=True)
    return x / jnp.maximum(n, eps)


def _build_o9(X):
    B, N, _ = X.shape
    dX = X[:, 1:, :] - X[:, -1:, :]
    U = _normalize(dX, -1)
    u_2 = U[:, :-2, :]
    u_1 = U[:, 1:-1, :]
    n_2 = _normalize(jnp.cross(u_2, u_1), -1)
    o_1 = _normalize(u_2 - u_1, -1)
    O = jnp.stack([o_1, n_2, jnp.cross(o_1, n_2)], -1)
    O = O.reshape(B, N - 3, 9)
    return jnp.pad(O, ((0, 0), (1, 2), (0, 0)))


def _knn_body(x_ref, xt_ref, idx_ref, dn_ref, *, R, N):
    xq = x_ref[0]
    xt = xt_ref[0]
    dx = xq[:, 0:1] - xt[0:1, :]
    dy = xq[:, 1:2] - xt[1:2, :]
    dz = xq[:, 2:3] - xt[2:3, :]
    s = dx * dx + dy * dy + dz * dz
    d = jnp.sqrt(s + 1e-6)
    iota = lax.broadcasted_iota(jnp.int32, (R, N), 1)
    for j in range(K_NEIGH):
        m = jnp.min(d, axis=1, keepdims=True)
        sel = d == m
        iv = jnp.min(jnp.where(sel, iota, N), axis=1, keepdims=True)
        idx_ref[0, :, j:j + 1] = iv
        dn_ref[0, :, j:j + 1] = m
        d = jnp.where(iota == iv, jnp.float32(jnp.inf), d)


def _knn(X):
    B, N, _ = X.shape
    R = 512 if N % 512 == 0 else 8
    XT = jnp.swapaxes(X, 1, 2)
    body = functools.partial(_knn_body, R=R, N=N)
    return pl.pallas_call(
        body,
        grid=(B, N // R),
        in_specs=[
            pl.BlockSpec((1, R, 3), lambda b, n: (b, n, 0)),
            pl.BlockSpec((1, 3, N), lambda b, n: (b, 0, 0)),
        ],
        out_specs=[
            pl.BlockSpec((1, R, K_NEIGH), lambda b, n: (b, n, 0)),
            pl.BlockSpec((1, R, K_NEIGH), lambda b, n: (b, n, 0)),
        ],
        out_shape=[
            jax.ShapeDtypeStruct((B, N, K_NEIGH), jnp.int32),
            jax.ShapeDtypeStruct((B, N, K_NEIGH), jnp.float32),
        ],
    )(X, XT)


_FC = 12


def _sc_gather(tbl2, idxg):
    BNK = idxg.shape[0]
    V = tbl2.shape[0] // _FC
    NC, NS = 2, 16
    NW = NC * NS
    bpw = BNK // NW
    CH = 1920 if bpw % 1920 == 0 else bpw
    NCH = bpw // CH
    NG = CH // 16
    mesh = plsc.VectorSubcoreMesh(core_axis_name="c", subcore_axis_name="s",
                                  num_cores=NC, num_subcores=NS)

    @functools.partial(
        pl.kernel, mesh=mesh,
        out_type=jax.ShapeDtypeStruct((BNK * _FC,), jnp.float32),
        compiler_params=pltpu.CompilerParams(needs_layout_passes=False),
        scratch_types=[
            pltpu.VMEM((V * _FC,), jnp.float32),
            pltpu.VMEM((CH,), jnp.int32),
            pltpu.VMEM((CH * _FC,), jnp.float32),
        ],
    )
    def k(tbl_hbm, idx_hbm, out_hbm, tbl_v, idx_v, rows_v):
        wid = lax.axis_index("s") * NC + lax.axis_index("c")
        base = wid * bpw
        pltpu.sync_copy(tbl_hbm, tbl_v)
        iota16 = lax.iota(jnp.int32, 16)

        def chunk(c, carry):
            off = base + c * CH
            pltpu.sync_copy(idx_hbm.at[pl.ds(off, CH)], idx_v)

            def group(g, carry2):
                idx16 = idx_v[pl.ds(g * 16, 16)]
                src = idx16 * _FC
                dst = (iota16 + g * 16) * _FC
                for col in range(_FC):
                    v = plsc.load_gather(tbl_v, [src + col])
                    plsc.store_scatter(rows_v, [dst + col], v)
                return carry2

            lax.fori_loop(0, NG, group, 0)
            pltpu.sync_copy(rows_v, out_hbm.at[pl.ds(off * _FC, CH * _FC)])
            return carry

        lax.fori_loop(0, NCH, chunk, 0)

    return k(tbl2, idxg)


def _np_g(rows, cols, pairs):
    g = np.zeros((rows, cols), np.float32)
    for r, c in pairs:
        g[r, c] = 1.0
    return g

_G9 = _np_g(9, 3, [(j, j // 3) for j in range(9)])
_G27 = _np_g(27, 9, [(9 * c + 3 * a + b, 3 * a + b)
                     for c in range(3) for a in range(3) for b in range(3)])
_S9 = _np_g(3, 9, [(l % 3, l) for l in range(9)])
_S27A = _np_g(9, 27, [(3 * (l // 9) + (l % 9) // 3, l) for l in range(27)])
_S27B = _np_g(9, 27, [(3 * (l // 9) + l % 3, l) for l in range(27)])
_C9 = np.zeros((9, 7), np.float32)
_C9[0, 0], _C9[4, 0], _C9[8, 0] = 1, -1, -1
_C9[0, 1], _C9[4, 1], _C9[8, 1] = -1, 1, -1
_C9[0, 2], _C9[4, 2], _C9[8, 2] = -1, -1, 1
_C9[7, 3], _C9[5, 3] = 1, -1
_C9[2, 4], _C9[6, 4] = 1, -1
_C9[3, 5], _C9[1, 5] = 1, -1
_C9[0, 6] = _C9[4, 6] = _C9[8, 6] = 1

_FREQ = np.exp(np.arange(0, NUM_PE, 2, dtype=np.float32)
               * (-(np.log(10000.0) / NUM_PE))).reshape(1, NUM_PE // 2)
_RBF_MU = np.linspace(0.0, 20.0, NUM_RBF, dtype=np.float32).reshape(1, NUM_RBF)
_RBF_SIG = (20.0 - 0.0) / NUM_RBF


def _edge_body(nf_ref, qf_ref, dpe_ref, dn_ref, wt_ref, b_ref, g_ref, bt_ref,
               g9_ref, g27_ref, c9_ref, freq_ref, rbfmu_ref,
               s9_ref, s27a_ref, s27b_ref, out_ref, *, E):
    nf = nf_ref[...]
    qf = qf_ref[...]
    xi = qf[:, 0:3]
    oi = qf[:, 3:12]
    xn = nf[:, 0:3]
    on = nf[:, 3:12]
    dxn = xn - xi
    f32 = jnp.float32
    p9 = oi * jnp.dot(dxn, s9_ref[...], preferred_element_type=f32)
    du_raw = jnp.dot(p9, g9_ref[...], preferred_element_type=f32)
    dun = jnp.sqrt(jnp.sum(du_raw * du_raw, axis=1, keepdims=True))
    du = du_raw / jnp.maximum(dun, 1e-12)
    a27 = jnp.dot(oi, s27a_ref[...], preferred_element_type=f32)
    b27 = jnp.dot(on, s27b_ref[...], preferred_element_type=f32)
    r9 = jnp.dot(a27 * b27, g27_ref[...], preferred_element_type=f32)
    t = jnp.dot(r9, c9_ref[...], preferred_element_type=jnp.float32)
    mag = 0.5 * jnp.sqrt(jnp.abs(1.0 + t[:, 0:3]))
    xyz = jnp.sign(t[:, 3:6]) * mag
    w = jnp.sqrt(jnp.maximum(1.0 + t[:, 6:7], 0.0)) / 2.0
    q = jnp.concatenate([xyz, w], axis=1)
    qn = jnp.sqrt(jnp.sum(q * q, axis=1, keepdims=True))
    q = q / jnp.maximum(qn, 1e-12)
    ang = dpe_ref[...] * freq_ref[...]
    rb = (dn_ref[...] - rbfmu_ref[...]) / _RBF_SIG
    rbf = jnp.exp(-(rb * rb))
    f = jnp.concatenate(
        [jnp.cos(ang), jnp.sin(ang), rbf, du, q, jnp.zeros((E, 1), jnp.float32)],
        axis=1)
    e = jnp.dot(f, wt_ref[...], preferred_element_type=jnp.float32) + b_ref[...]
    mu = jnp.mean(e, axis=1, keepdims=True)
    var = jnp.mean((e - mu) * (e - mu), axis=1, keepdims=True)
    out_ref[...] = g_ref[...] * (e - mu) / jnp.sqrt(var + 1e-6) + bt_ref[...]


def _edges(nf, qf, dpe, dn, Wt, b, gamma, beta):
    BNK = nf.shape[0]
    E = 2048 if BNK % 2048 == 0 else (512 if BNK % 512 == 0 else 480)
    D = Wt.shape[1]
    body = functools.partial(_edge_body, E=E)
    return pl.pallas_call(
        body,
        grid=(BNK // E,),
        in_specs=[
            pl.BlockSpec((E, _FC), lambda i: (i, 0)),
            pl.BlockSpec((E, _FC), lambda i: (i, 0)),
            pl.BlockSpec((E, 1), lambda i: (i, 0)),
            pl.BlockSpec((E, 1), lambda i: (i, 0)),
            pl.BlockSpec((40, D), lambda i: (0, 0)),
            pl.BlockSpec((1, D), lambda i: (0, 0)),
            pl.BlockSpec((1, D), lambda i: (0, 0)),
            pl.BlockSpec((1, D), lambda i: (0, 0)),
            pl.BlockSpec((9, 3), lambda i: (0, 0)),
            pl.BlockSpec((27, 9), lambda i: (0, 0)),
            pl.BlockSpec((9, 7), lambda i: (0, 0)),
            pl.BlockSpec((1, NUM_PE // 2), lambda i: (0, 0)),
            pl.BlockSpec((1, NUM_RBF), lambda i: (0, 0)),
            pl.BlockSpec((3, 9), lambda i: (0, 0)),
            pl.BlockSpec((9, 27), lambda i: (0, 0)),
            pl.BlockSpec((9, 27), lambda i: (0, 0)),
        ],
        out_specs=pl.BlockSpec((E, D), lambda i: (i, 0)),
        out_shape=jax.ShapeDtypeStruct((BNK, D), jnp.float32),
    )(nf, qf, dpe, dn, Wt, b, gamma, beta,
      jnp.asarray(_G9), jnp.asarray(_G27), jnp.asarray(_C9),
      jnp.asarray(_FREQ), jnp.asarray(_RBF_MU),
      jnp.asarray(_S9), jnp.asarray(_S27A), jnp.asarray(_S27B))


def kernel(X, mask, W, b, gamma, beta):
    B, N, _ = X.shape
    K = K_NEIGH
    D = W.shape[0]
    E_idx, Dn = _knn(X)
    O9 = _build_o9(X)
    tbl = jnp.concatenate([X, O9], axis=-1)
    offs = (jnp.arange(B, dtype=jnp.int32) * N).reshape(B, 1, 1)
    idx_g = (E_idx + offs).reshape(-1)
    nf = _sc_gather(tbl.reshape(-1), idx_g).reshape(-1, _FC)
    qf = jnp.broadcast_to(tbl[:, :, None, :], (B, N, K, _FC)).reshape(-1, _FC)
    ii = jnp.arange(N, dtype=jnp.float32).reshape(1, N, 1)
    dpe = (E_idx.astype(jnp.float32) - ii).reshape(-1, 1)
    dnf = Dn.reshape(-1, 1)
    Wt = jnp.zeros((40, D), jnp.float32).at[:W.shape[1]].set(W.T)
    E_flat = _edges(nf, qf, dpe, dnf, Wt,
                    b.reshape(1, D), gamma.reshape(1, D), beta.reshape(1, D))
    return E_flat.reshape(B, N, K, D), E_idx
